# baseline (speedup 1.0000x reference)
import functools
import math

import jax
import jax.numpy as jnp
from jax import lax
from jax.experimental import pallas as pl
from jax.experimental.pallas import tpu as pltpu
from jax.experimental.pallas import tpu_sc as plsc

_N = 100000
_F = 128
_K = 32
_NC = 2
_NS = 16
_NW = _NC * _NS
_BLK = 128
_NFULL = _N // _BLK
_TAIL = _N - _NFULL * _BLK
_TAIL0 = _NFULL * _BLK
_TAIL_W = _NFULL % _NW


def _rsqrt_vec(x):
    i = plsc.bitcast(x, jnp.int32)
    i = jnp.int32(0x5F3759DF) - (i >> 1)
    y = plsc.bitcast(i, jnp.float32)
    for _ in range(3):
        y = y * (1.5 - 0.5 * x * y * y)
    return y


def _tanh_scaled(k, two_inv):
    a = jnp.minimum(k * two_inv, 40.0)
    e = jnp.exp(a)
    return (e - 1.0) / (e + 1.0)


def _sort16(k, v, desc):
    return plsc.sort_key_val(k, v, descending=desc)


def _merge16(a, b, out_desc):
    (ka, va), (kb, vb) = a, b
    m = ka >= kb
    hk = jnp.maximum(ka, kb)
    hv = jnp.where(m, va, vb)
    lk = jnp.minimum(ka, kb)
    lv = jnp.where(m, vb, va)
    if out_desc:
        return _sort16(hk, hv, True), _sort16(lk, lv, True)
    return _sort16(lk, lv, False), _sort16(hk, hv, False)


def _merge32_top(A, B, out_desc):
    (ahk, ahv), (alk, alv) = A
    (blk, blv), (bhk, bhv) = B
    m1 = ahk >= blk
    m1k = jnp.maximum(ahk, blk)
    m1v = jnp.where(m1, ahv, blv)
    m2 = alk >= bhk
    m2k = jnp.maximum(alk, bhk)
    m2v = jnp.where(m2, alv, bhv)
    mm = m1k >= m2k
    hk = jnp.maximum(m1k, m2k)
    hv = jnp.where(mm, m1v, m2v)
    lk = jnp.minimum(m1k, m2k)
    lv = jnp.where(mm, m2v, m1v)
    if out_desc:
        return _sort16(hk, hv, True), _sort16(lk, lv, True)
    return _sort16(lk, lv, False), _sort16(hk, hv, False)


def _row_topk(evecs, svecs, two_inv):
    ch = [_sort16(evecs[c] * svecs[c], evecs[c], desc=(c % 2 == 0))
          for c in range(8)]
    A = _merge16(ch[0], ch[1], True)
    B = _merge16(ch[2], ch[3], False)
    C = _merge16(ch[4], ch[5], True)
    D = _merge16(ch[6], ch[7], False)
    E = _merge32_top(A, B, True)
    F = _merge32_top(C, D, False)
    (ghk, ghv), (glk, glv) = _merge32_top(E, F, True)
    o_hi = ghv * _tanh_scaled(ghk, two_inv)
    o_lo = glv * _tanh_scaled(glk, two_inv)
    return o_hi, o_lo


@functools.partial(
    pl.kernel,
    out_type=jax.ShapeDtypeStruct((_K, _N), jnp.float32),
    mesh=plsc.VectorSubcoreMesh(
        core_axis_name="c", subcore_axis_name="s",
        num_cores=_NC, num_subcores=_NS),
    scratch_types=[
        pltpu.VMEM((_F,), jnp.float32),
        pltpu.VMEM((2, _BLK, _F), jnp.float32),
        pltpu.VMEM((2, _K, _BLK), jnp.float32),
        pltpu.VMEM((_TAIL, _F), jnp.float32),
        pltpu.VMEM((_K, _TAIL), jnp.float32),
        pltpu.SemaphoreType.DMA((2,)),
        pltpu.SemaphoreType.DMA((2,)),
    ],
    compiler_params=pltpu.CompilerParams(needs_layout_passes=False),
)
def _topk_sc(emb_hbm, scorer_hbm, out_hbm, scorer_v, emb_v, out_v,
             emb_t, out_t, sem_in, sem_out):
    wid = lax.axis_index("s") * _NC + lax.axis_index("c")
    nblk_w = (_NFULL + _NW - 1 - wid) // _NW

    pltpu.sync_copy(scorer_hbm, scorer_v)
    svecs = [scorer_v[pl.ds(16 * c, 16)] for c in range(8)]
    acc = svecs[0] * svecs[0]
    for c in range(1, 8):
        acc = acc + svecs[c] * svecs[c]
    total = jnp.sum(acc)
    two_inv = 2.0 * _rsqrt_vec(lax.broadcast_in_dim(total, (16,), ()))

    def _in_copy(i, buf):
        row0 = (wid + i * _NW) * _BLK
        return pltpu.make_async_copy(
            emb_hbm.at[pl.ds(row0, _BLK)],
            emb_v.at[buf], sem_in.at[buf])

    def _out_copy(i, buf):
        row0 = (wid + i * _NW) * _BLK
        return pltpu.make_async_copy(
            out_v.at[buf], out_hbm.at[:, pl.ds(row0, _BLK)],
            sem_out.at[buf])

    _in_copy(0, 0).start()

    k_lo = lax.iota(jnp.int32, 16)
    k_hi = k_lo + 16

    def blk_body(i, carry):
        par = lax.rem(i, 2)
        nxt = 1 - par
        par_vec = lax.broadcast_in_dim(par, (16,), ())

        @pl.when(i + 1 < nblk_w)
        def _():
            _in_copy(i + 1, nxt).start()

        _in_copy(i, par).wait()

        @pl.when(i >= 2)
        def _():
            _out_copy(i - 2, par).wait()

        @plsc.parallel_loop(0, _BLK, unroll=2)
        def row_body(r):
            evecs = [emb_v[par, r, pl.ds(16 * c, 16)] for c in range(8)]
            o_hi, o_lo = _row_topk(evecs, svecs, two_inv)
            r_vec = lax.broadcast_in_dim(r, (16,), ())
            plsc.store_scatter(out_v, [par_vec, k_lo, r_vec], o_hi)
            plsc.store_scatter(out_v, [par_vec, k_hi, r_vec], o_lo)

        _out_copy(i, par).start()
        return carry

    lax.fori_loop(0, nblk_w, blk_body, 0)
    _out_copy(nblk_w - 2, lax.rem(nblk_w - 2, 2)).wait()
    _out_copy(nblk_w - 1, lax.rem(nblk_w - 1, 2)).wait()

    @pl.when(wid == _TAIL_W)
    def _():
        pltpu.sync_copy(emb_hbm.at[pl.ds(_TAIL0, _TAIL)], emb_t)

        @plsc.parallel_loop(0, _TAIL, unroll=2)
        def tail_row(r):
            evecs = [emb_t[r, pl.ds(16 * c, 16)] for c in range(8)]
            o_hi, o_lo = _row_topk(evecs, svecs, two_inv)
            r_vec = lax.broadcast_in_dim(r, (16,), ())
            plsc.store_scatter(out_t, [k_lo, r_vec], o_hi)
            plsc.store_scatter(out_t, [k_hi, r_vec], o_lo)

        pltpu.sync_copy(out_t, out_hbm.at[:, pl.ds(_TAIL0, _TAIL)])


def kernel(node_embs, scorer):
    return _topk_sc(node_embs, scorer).T


# speedup vs baseline: 1.1784x; 1.1784x over previous
import functools
import math

import jax
import jax.numpy as jnp
from jax import lax
from jax.experimental import pallas as pl
from jax.experimental.pallas import tpu as pltpu
from jax.experimental.pallas import tpu_sc as plsc

_N = 100000
_F = 128
_K = 32
_NC = 2
_NS = 16
_NW = _NC * _NS
_BLK = 200
_NBLK = _N // _BLK


def _rsqrt_vec(x):
    i = plsc.bitcast(x, jnp.int32)
    i = jnp.int32(0x5F3759DF) - (i >> 1)
    y = plsc.bitcast(i, jnp.float32)
    for _ in range(3):
        y = y * (1.5 - 0.5 * x * y * y)
    return y


def _tanh_scaled(k, two_inv):
    a = jnp.minimum(k * two_inv, 40.0)
    e = jnp.exp(a)
    return (e - 1.0) / (e + 1.0)


def _sort16(k, v, desc):
    return plsc.sort_key_val(k, v, descending=desc)


def _merge16(a, b, out_desc):
    (ka, va), (kb, vb) = a, b
    m = ka >= kb
    hk = jnp.maximum(ka, kb)
    hv = jnp.where(m, va, vb)
    lk = jnp.minimum(ka, kb)
    lv = jnp.where(m, vb, va)
    if out_desc:
        return _sort16(hk, hv, True), _sort16(lk, lv, True)
    return _sort16(lk, lv, False), _sort16(hk, hv, False)


def _merge32_top(A, B, out_desc):
    (ahk, ahv), (alk, alv) = A
    (blk, blv), (bhk, bhv) = B
    m1 = ahk >= blk
    m1k = jnp.maximum(ahk, blk)
    m1v = jnp.where(m1, ahv, blv)
    m2 = alk >= bhk
    m2k = jnp.maximum(alk, bhk)
    m2v = jnp.where(m2, alv, bhv)
    mm = m1k >= m2k
    hk = jnp.maximum(m1k, m2k)
    hv = jnp.where(mm, m1v, m2v)
    lk = jnp.minimum(m1k, m2k)
    lv = jnp.where(mm, m2v, m1v)
    if out_desc:
        return _sort16(hk, hv, True), _sort16(lk, lv, True)
    return _sort16(lk, lv, False), _sort16(hk, hv, False)


def _row_topk(evecs, svecs, two_inv):
    ch = [_sort16(evecs[c] * svecs[c], evecs[c], desc=(c % 2 == 0))
          for c in range(8)]
    A = _merge16(ch[0], ch[1], True)
    B = _merge16(ch[2], ch[3], False)
    C = _merge16(ch[4], ch[5], True)
    D = _merge16(ch[6], ch[7], False)
    E = _merge32_top(A, B, True)
    F = _merge32_top(C, D, False)
    (ghk, ghv), (glk, glv) = _merge32_top(E, F, True)
    o_hi = ghv * _tanh_scaled(ghk, two_inv)
    o_lo = glv * _tanh_scaled(glk, two_inv)
    return o_hi, o_lo


@functools.partial(
    pl.kernel,
    out_type=jax.ShapeDtypeStruct((_N, _K), jnp.float32),
    mesh=plsc.VectorSubcoreMesh(
        core_axis_name="c", subcore_axis_name="s",
        num_cores=_NC, num_subcores=_NS),
    scratch_types=[
        pltpu.VMEM((_F,), jnp.float32),
        pltpu.VMEM((2, _BLK, _F), jnp.float32),
        pltpu.VMEM((2, _BLK, _K), jnp.float32),
        pltpu.SemaphoreType.DMA((2,)),
        pltpu.SemaphoreType.DMA((2,)),
    ],
    compiler_params=pltpu.CompilerParams(needs_layout_passes=False),
)
def _topk_sc(emb_hbm, scorer_hbm, out_hbm, scorer_v, emb_v, out_v,
             sem_in, sem_out):
    wid = lax.axis_index("s") * _NC + lax.axis_index("c")
    nblk_w = (_NBLK + _NW - 1 - wid) // _NW

    pltpu.sync_copy(scorer_hbm, scorer_v)
    svecs = [scorer_v[pl.ds(16 * c, 16)] for c in range(8)]
    acc = svecs[0] * svecs[0]
    for c in range(1, 8):
        acc = acc + svecs[c] * svecs[c]
    total = jnp.sum(acc)
    two_inv = 2.0 * _rsqrt_vec(lax.broadcast_in_dim(total, (16,), ()))

    def _in_copy(i, buf):
        row0 = (wid + i * _NW) * _BLK
        return pltpu.make_async_copy(
            emb_hbm.at[pl.ds(row0, _BLK)],
            emb_v.at[buf], sem_in.at[buf])

    def _out_copy(i, buf):
        row0 = (wid + i * _NW) * _BLK
        return pltpu.make_async_copy(
            out_v.at[buf], out_hbm.at[pl.ds(row0, _BLK)],
            sem_out.at[buf])

    _in_copy(0, 0).start()

    def blk_body(i, carry):
        par = lax.rem(i, 2)
        nxt = 1 - par

        @pl.when(i + 1 < nblk_w)
        def _():
            _in_copy(i + 1, nxt).start()

        _in_copy(i, par).wait()

        @pl.when(i >= 2)
        def _():
            _out_copy(i - 2, par).wait()

        @plsc.parallel_loop(0, _BLK, unroll=2)
        def row_body(r):
            evecs = [emb_v[par, r, pl.ds(16 * c, 16)] for c in range(8)]
            o_hi, o_lo = _row_topk(evecs, svecs, two_inv)
            out_v[par, r, pl.ds(0, 16)] = o_hi
            out_v[par, r, pl.ds(16, 16)] = o_lo

        _out_copy(i, par).start()
        return carry

    lax.fori_loop(0, nblk_w, blk_body, 0)
    _out_copy(nblk_w - 2, lax.rem(nblk_w - 2, 2)).wait()
    _out_copy(nblk_w - 1, lax.rem(nblk_w - 1, 2)).wait()


def kernel(node_embs, scorer):
    return _topk_sc(node_embs, scorer)


# speedup vs baseline: 1.1789x; 1.0004x over previous
import functools

import jax
import jax.numpy as jnp
from jax import lax
from jax.experimental import pallas as pl
from jax.experimental.pallas import tpu as pltpu
from jax.experimental.pallas import tpu_sc as plsc

_N = 100000
_F = 128
_K = 32
_NC = 2
_NS = 16
_NW = _NC * _NS
_BLK = 200
_NBLK = _N // _BLK


def _rsqrt_vec(x):
    i = plsc.bitcast(x, jnp.int32)
    i = jnp.int32(0x5F3759DF) - (i >> 1)
    y = plsc.bitcast(i, jnp.float32)
    for _ in range(3):
        y = y * (1.5 - 0.5 * x * y * y)
    return y


def _tanh_scaled(k, two_inv):
    a = jnp.minimum(k * two_inv, 40.0)
    e = jnp.exp(a)
    return (e - 1.0) / (e + 1.0)


def _sort16(k, v, desc):
    return plsc.sort_key_val(k, v, descending=desc)


def _merge16(a, b, out_desc):
    (ka, va), (kb, vb) = a, b
    m = ka >= kb
    hk = jnp.maximum(ka, kb)
    hv = jnp.where(m, va, vb)
    lk = jnp.minimum(ka, kb)
    lv = jnp.where(m, vb, va)
    if out_desc:
        return _sort16(hk, hv, True), _sort16(lk, lv, True)
    return _sort16(lk, lv, False), _sort16(hk, hv, False)


def _merge32_top(A, B, out_desc):
    (ahk, ahv), (alk, alv) = A
    (blk, blv), (bhk, bhv) = B
    m1 = ahk >= blk
    m1k = jnp.maximum(ahk, blk)
    m1v = jnp.where(m1, ahv, blv)
    m2 = alk >= bhk
    m2k = jnp.maximum(alk, bhk)
    m2v = jnp.where(m2, alv, bhv)
    mm = m1k >= m2k
    hk = jnp.maximum(m1k, m2k)
    hv = jnp.where(mm, m1v, m2v)
    lk = jnp.minimum(m1k, m2k)
    lv = jnp.where(mm, m2v, m1v)
    if out_desc:
        return _sort16(hk, hv, True), _sort16(lk, lv, True)
    return _sort16(lk, lv, False), _sort16(hk, hv, False)


def _row_topk(evecs, svecs, two_inv):
    ch = [_sort16(evecs[c] * svecs[c], evecs[c], desc=(c % 2 == 0))
          for c in range(8)]
    A = _merge16(ch[0], ch[1], True)
    B = _merge16(ch[2], ch[3], False)
    C = _merge16(ch[4], ch[5], True)
    D = _merge16(ch[6], ch[7], False)
    E = _merge32_top(A, B, True)
    F = _merge32_top(C, D, False)
    (ghk, ghv), (glk, glv) = _merge32_top(E, F, True)
    o_hi = ghv * _tanh_scaled(ghk, two_inv)
    o_lo = glv * _tanh_scaled(glk, two_inv)
    return o_hi, o_lo


@functools.partial(
    pl.kernel,
    out_type=jax.ShapeDtypeStruct((_N, _K), jnp.float32),
    mesh=plsc.VectorSubcoreMesh(
        core_axis_name="c", subcore_axis_name="s",
        num_cores=_NC, num_subcores=_NS),
    scratch_types=[
        pltpu.VMEM((_F,), jnp.float32),
        pltpu.VMEM((2, _BLK, _F), jnp.float32),
        pltpu.VMEM((2, _BLK, _K), jnp.float32),
        pltpu.SemaphoreType.DMA((2,)),
        pltpu.SemaphoreType.DMA((2,)),
    ],
    compiler_params=pltpu.CompilerParams(needs_layout_passes=False),
)
def _topk_sc(emb_hbm, scorer_hbm, out_hbm, scorer_v, emb_v, out_v,
             sem_in, sem_out):
    wid = lax.axis_index("s") * _NC + lax.axis_index("c")
    nblk_w = (_NBLK + _NW - 1 - wid) // _NW

    pltpu.sync_copy(scorer_hbm, scorer_v)
    svecs = [scorer_v[pl.ds(16 * c, 16)] for c in range(8)]
    acc = svecs[0] * svecs[0]
    for c in range(1, 8):
        acc = acc + svecs[c] * svecs[c]
    total = jnp.sum(acc)
    two_inv = 2.0 * _rsqrt_vec(lax.broadcast_in_dim(total, (16,), ()))

    def _in_copy(i, buf):
        row0 = (wid + i * _NW) * _BLK
        return pltpu.make_async_copy(
            emb_hbm.at[pl.ds(row0, _BLK)],
            emb_v.at[buf], sem_in.at[buf])

    def _out_copy(i, buf):
        row0 = (wid + i * _NW) * _BLK
        return pltpu.make_async_copy(
            out_v.at[buf], out_hbm.at[pl.ds(row0, _BLK)],
            sem_out.at[buf])

    _in_copy(0, 0).start()

    def blk_body(i, carry):
        par = lax.rem(i, 2)
        nxt = 1 - par

        @pl.when(i + 1 < nblk_w)
        def _():
            _in_copy(i + 1, nxt).start()

        _in_copy(i, par).wait()

        @pl.when(i >= 2)
        def _():
            _out_copy(i - 2, par).wait()

        @plsc.parallel_loop(0, _BLK, unroll=2)
        def row_body(r):
            evecs = [emb_v[par, r, pl.ds(16 * c, 16)] for c in range(8)]
            o_hi, o_lo = _row_topk(evecs, svecs, two_inv)
            out_v[par, r, pl.ds(0, 16)] = o_hi
            out_v[par, r, pl.ds(16, 16)] = o_lo

        _out_copy(i, par).start()
        return carry

    lax.fori_loop(0, nblk_w, blk_body, 0)
    _out_copy(nblk_w - 2, lax.rem(nblk_w - 2, 2)).wait()
    _out_copy(nblk_w - 1, lax.rem(nblk_w - 1, 2)).wait()


def kernel(node_embs, scorer):
    return _topk_sc(node_embs, scorer)
